# Initial kernel scaffold; baseline (speedup 1.0000x reference)
import jax
import jax.numpy as jnp
from jax.experimental import pallas as pl


def kernel(x, W):
    raise NotImplementedError("write your pallas kernel here")


# speedup vs baseline: 1.0733x; 1.0733x over previous
import functools

import jax
import jax.numpy as jnp
from jax import lax
from jax.experimental import pallas as pl
from jax.experimental.pallas import tpu as pltpu
from jax.experimental.pallas import tpu_sc as plsc

N_FIELDS = 26
VOCAB = 100
DIM = 128
BATCH = 16384

NC = 2
NS = 16
NW = NC * NS
L = 16

ROWS = BATCH * N_FIELDS
ROWS_PER_W = ROWS // NW
CHUNK = 128
ITERS = ROWS_PER_W // CHUNK


def _sc_gather(xflat, U):
  mesh = plsc.VectorSubcoreMesh(core_axis_name="c", subcore_axis_name="s")

  @functools.partial(
      pl.kernel,
      mesh=mesh,
      out_type=jax.ShapeDtypeStruct((ROWS, DIM), jnp.float32),
      scratch_types=[
          pltpu.VMEM((CHUNK,), jnp.int32),
          pltpu.VMEM((CHUNK, DIM), jnp.float32),
          pltpu.SemaphoreType.DMA,
      ],
  )
  def k(x_hbm, u_hbm, e_hbm, idx_v, rows_v, sem):
    wid = lax.axis_index("s") * NC + lax.axis_index("c")
    base = wid * ROWS_PER_W
    lane = lax.iota(jnp.int32, L)

    def body(t, carry):
      off = base + t * CHUNK
      pltpu.sync_copy(x_hbm.at[pl.ds(off, CHUNK)], idx_v)
      for j in range(CHUNK // L):
        pos = off + j * L + lane
        fld = lax.rem(pos, N_FIELDS)
        sl = pl.ds(j * L, L)
        idx_v[sl] = idx_v[sl] + fld * VOCAB
      pltpu.async_copy(u_hbm.at[idx_v], rows_v, sem).wait()
      pltpu.sync_copy(rows_v, e_hbm.at[pl.ds(off, CHUNK)])
      return carry

    lax.fori_loop(0, ITERS, body, 0)

  return k(xflat, U)


def _tc_transpose(E):
  blk = 128

  def body(e_ref, o_ref):
    o_ref[...] = jnp.transpose(e_ref[...], (0, 2, 1))

  return pl.pallas_call(
      body,
      grid=(BATCH // blk,),
      in_specs=[pl.BlockSpec((blk, N_FIELDS, DIM), lambda i: (i, 0, 0))],
      out_specs=pl.BlockSpec((blk, DIM, N_FIELDS), lambda i: (i, 0, 0)),
      out_shape=jax.ShapeDtypeStruct((BATCH, DIM, N_FIELDS), jnp.float32),
  )(E)


def kernel(x, W):
  xflat = x.reshape(-1).astype(jnp.int32)
  U = W.reshape(N_FIELDS * VOCAB, DIM)
  E = _sc_gather(xflat, U)
  return _tc_transpose(E.reshape(BATCH, N_FIELDS, DIM))
